# baseline (speedup 1.0000x reference)
import functools
import jax
import jax.numpy as jnp
from jax import lax
from jax.experimental import pallas as pl
from jax.experimental.pallas import tpu as pltpu

B, T, D = 16, 20000, 128
KERNEL = 5
TOPK = 100
UNIT = 2.0
TB = 800
NT = T // TB


RB = 8192
NR = -(-(B * T) // RB)


def _proj_body(w_ref, b_ref, x_ref, sal_ref, c_ref, win_ref, off_ref):
    xr = x_ref[:]
    w = w_ref[:]
    y = lax.dot_general(w, xr, (((1,), (1,)), ((), ())),
                        precision=lax.Precision.DEFAULT,
                        preferred_element_type=jnp.float32)
    y = y + b_ref[:]
    mask = jnp.where(sal_ref[:] >= 0.0, 1.0, 0.0)
    c_ref[:] = jax.nn.sigmoid(y[0]) * mask
    win_ref[:] = y[1]
    off_ref[:] = y[2]


@jax.jit
def _project(x, sal, w8, b8):
    out = jax.ShapeDtypeStruct((B * T,), jnp.float32)
    return pl.pallas_call(
        _proj_body,
        grid=(NR,),
        in_specs=[
            pl.BlockSpec((8, D), lambda r: (0, 0)),
            pl.BlockSpec((8, 1), lambda r: (0, 0)),
            pl.BlockSpec((RB, D), lambda r: (r, 0)),
            pl.BlockSpec((RB,), lambda r: (r,)),
        ],
        out_specs=[
            pl.BlockSpec((RB,), lambda r: (r,)),
            pl.BlockSpec((RB,), lambda r: (r,)),
            pl.BlockSpec((RB,), lambda r: (r,)),
        ],
        out_shape=[out, out, out],
    )(w8, b8, x.reshape(B * T, D), sal.reshape(B * T))


NC = 160
CL = 128
TP = NC * CL


def _shift(a, s, fill):
    if s == 0:
        return a
    if s > 0:
        return jnp.concatenate(
            [a[:, s:], jnp.full((B, s), fill, a.dtype)], axis=1)
    return jnp.concatenate(
        [jnp.full((B, -s), fill, a.dtype), a[:, :s]], axis=1)


def _prep_body(c_ref, w_ref, o_ref, pwo_ref, g_ref):
    c = c_ref[:]
    hm = c
    for s in (-2, -1, 1, 2):
        hm = jnp.maximum(hm, _shift(c, s, -jnp.inf))
    p = jnp.where(hm == c, c, 0.0)
    pad = jnp.full((B, TP - T), -1.0, jnp.float32)
    p3 = jnp.concatenate([p, pad], axis=1).reshape(B, NC, CL)
    w3 = jnp.concatenate([w_ref[:], pad], axis=1).reshape(B, NC, CL)
    o3 = jnp.concatenate([o_ref[:], pad], axis=1).reshape(B, NC, CL)
    pwo_ref[:] = jnp.concatenate(
        [p3.reshape(B * NC, CL), w3.reshape(B * NC, CL),
         o3.reshape(B * NC, CL)], axis=1)
    g_ref[:] = jnp.max(p3, axis=2)


@jax.jit
def _prep(c, w, o):
    return pl.pallas_call(
        _prep_body,
        out_shape=[jax.ShapeDtypeStruct((B * NC, 3 * CL), jnp.float32),
                   jax.ShapeDtypeStruct((B, NC), jnp.float32)],
    )(c, w, o)


def _select_body(pwo_ref, g_ref, b0_ref, b1_ref, sc_ref):
    iota_c = lax.broadcasted_iota(jnp.int32, (B, NC), 1)
    iota_l = lax.broadcasted_iota(jnp.int32, (B, CL), 1)
    iota_f = lax.broadcasted_iota(jnp.int32, (B, B * NC), 1)
    rowbase = lax.broadcasted_iota(jnp.int32, (B, 1), 0) * NC
    iota_k = lax.broadcasted_iota(jnp.int32, (1, TOPK), 1)
    pwo = pwo_ref[:]

    def step(r, carry):
        g, mprev, cprev, lprev, b0a, b1a, sca = carry
        m = jnp.max(g, axis=1, keepdims=True)
        cc = jnp.min(jnp.where(g == m, iota_c, NC), axis=1, keepdims=True)
        oh = (iota_f == (rowbase + cc)).astype(jnp.float32)
        gath = lax.dot_general(oh, pwo, (((1,), (0,)), ((), ())),
                               precision=lax.Precision.HIGHEST,
                               preferred_element_type=jnp.float32)
        pch = gath[:, 0:CL]
        wch = gath[:, CL:2 * CL]
        och = gath[:, 2 * CL:3 * CL]
        excl = (cc == cprev) & (m == mprev)
        elm = (pch == m) & ~(excl & (iota_l <= lprev))
        li = jnp.min(jnp.where(elm, iota_l, CL), axis=1, keepdims=True)
        pick = iota_l == li
        offv = jnp.sum(jnp.where(pick, och, 0.0), axis=1, keepdims=True)
        winv = jnp.sum(jnp.where(pick, wch, 0.0), axis=1, keepdims=True)
        rem = jnp.where((pch < m) | ((pch == m) & (iota_l > li)), pch, -1.0)
        nm = jnp.max(rem, axis=1, keepdims=True)
        g = jnp.where(iota_c == cc, nm, g)
        idxf = (cc * CL + li).astype(jnp.float32)
        center = jnp.clip(idxf + offv, 0.0, T - 1)
        winv = jnp.clip(winv, 0.0, None)
        b0 = jnp.clip(center - winv * 0.5, 0.0, T - 1) * UNIT
        b1 = jnp.clip(center + winv * 0.5, 0.0, T - 1) * UNIT + UNIT
        sel = (iota_k == r).astype(jnp.float32)
        return (g, m, cc, li,
                b0a + b0 * sel, b1a + b1 * sel, sca + m * sel)

    z = jnp.zeros((B, TOPK), jnp.float32)
    init = (g_ref[:], jnp.full((B, 1), -2.0, jnp.float32),
            jnp.full((B, 1), -1, jnp.int32), jnp.full((B, 1), -1, jnp.int32),
            z, z, z)
    out = lax.fori_loop(0, TOPK, step, init)
    b0_ref[:] = out[4]
    b1_ref[:] = out[5]
    sc_ref[:] = out[6]


@jax.jit
def _select(pwo, g):
    out = jax.ShapeDtypeStruct((B, TOPK), jnp.float32)
    return pl.pallas_call(
        _select_body,
        out_shape=[out, out, out],
    )(pwo, g)


@jax.jit
def _boundary(c, w, o):
    pwo, g = _prep(c, w, o)
    return _select(pwo, g)


def kernel(x, saliency, center_w, center_b, window_w, window_b,
           offset_w, offset_b):
    w8 = jnp.zeros((8, D), jnp.float32)
    w8 = w8.at[0].set(center_w[:, 0]).at[1].set(window_w[:, 0])
    w8 = w8.at[2].set(offset_w[:, 0])
    b8 = jnp.zeros((8, 1), jnp.float32)
    b8 = b8.at[0, 0].set(center_b[0]).at[1, 0].set(window_b[0])
    b8 = b8.at[2, 0].set(offset_b[0])
    c, w, o = _project(x, saliency, w8, b8)
    b0, b1, sc = _boundary(c.reshape(B, T), w.reshape(B, T), o.reshape(B, T))
    return jnp.stack([b0, b1, sc], axis=-1)


# speedup vs baseline: 1.5122x; 1.5122x over previous
import functools
import jax
import jax.numpy as jnp
from jax import lax
from jax.experimental import pallas as pl
from jax.experimental.pallas import tpu as pltpu

B, T, D = 16, 20000, 128
KERNEL = 5
TOPK = 100
UNIT = 2.0
TB = 800
NT = T // TB


RB = 8192
NR = -(-(B * T) // RB)


def _proj_body(w_ref, b_ref, x_ref, sal_ref, c_ref, win_ref, off_ref):
    xr = x_ref[:]
    w = w_ref[:]
    y = lax.dot_general(w, xr, (((1,), (1,)), ((), ())),
                        precision=lax.Precision.DEFAULT,
                        preferred_element_type=jnp.float32)
    y = y + b_ref[:]
    mask = jnp.where(sal_ref[:] >= 0.0, 1.0, 0.0)
    c_ref[:] = jax.nn.sigmoid(y[0]) * mask
    win_ref[:] = y[1]
    off_ref[:] = y[2]


@jax.jit
def _project(x, sal, w8, b8):
    out = jax.ShapeDtypeStruct((B * T,), jnp.float32)
    return pl.pallas_call(
        _proj_body,
        grid=(NR,),
        in_specs=[
            pl.BlockSpec((8, D), lambda r: (0, 0)),
            pl.BlockSpec((8, 1), lambda r: (0, 0)),
            pl.BlockSpec((RB, D), lambda r: (r, 0)),
            pl.BlockSpec((RB,), lambda r: (r,)),
        ],
        out_specs=[
            pl.BlockSpec((RB,), lambda r: (r,)),
            pl.BlockSpec((RB,), lambda r: (r,)),
            pl.BlockSpec((RB,), lambda r: (r,)),
        ],
        out_shape=[out, out, out],
    )(w8, b8, x.reshape(B * T, D), sal.reshape(B * T))


NC = 160
CL = 128
TP = NC * CL


def _shift(a, s, fill):
    if s == 0:
        return a
    if s > 0:
        return jnp.concatenate(
            [a[:, s:], jnp.full((B, s), fill, a.dtype)], axis=1)
    return jnp.concatenate(
        [jnp.full((B, -s), fill, a.dtype), a[:, :s]], axis=1)


def _boundary_body(c_ref, w_ref, o_ref, b0_ref, b1_ref, sc_ref, p_ref):
    c = c_ref[:]
    hm = c
    for s in (-2, -1, 1, 2):
        hm = jnp.maximum(hm, _shift(c, s, -jnp.inf))
    p_ref[:] = jnp.where(hm == c, c, 0.0)

    iota = lax.broadcasted_iota(jnp.int32, (B, T), 1)
    iota_k = lax.broadcasted_iota(jnp.int32, (1, TOPK), 1)
    wv = w_ref[:]
    ov = o_ref[:]

    def step(r, carry):
        mprev, iprev, b0a, b1a, sca = carry
        p = p_ref[:]
        live = (p < mprev) | ((p == mprev) & (iota > iprev))
        m = jnp.max(jnp.where(live, p, -1.0), axis=1, keepdims=True)
        idx = jnp.min(
            jnp.where((p == m) & ((m < mprev) | (iota > iprev)), iota, T),
            axis=1, keepdims=True)
        onehot = iota == idx
        off = jnp.sum(jnp.where(onehot, ov, 0.0), axis=1, keepdims=True)
        win = jnp.sum(jnp.where(onehot, wv, 0.0), axis=1, keepdims=True)
        center = jnp.clip(idx.astype(jnp.float32) + off, 0.0, T - 1)
        win = jnp.clip(win, 0.0, None)
        b0 = jnp.clip(center - win * 0.5, 0.0, T - 1) * UNIT
        b1 = jnp.clip(center + win * 0.5, 0.0, T - 1) * UNIT + UNIT
        sel = (iota_k == r).astype(jnp.float32)
        return (m, idx, b0a + b0 * sel, b1a + b1 * sel, sca + m * sel)

    z = jnp.zeros((B, TOPK), jnp.float32)
    init = (jnp.full((B, 1), 2.0, jnp.float32),
            jnp.full((B, 1), -1, jnp.int32), z, z, z)
    out = lax.fori_loop(0, TOPK, step, init)
    b0_ref[:] = out[2]
    b1_ref[:] = out[3]
    sc_ref[:] = out[4]


@jax.jit
def _boundary(c, w, o):
    out = jax.ShapeDtypeStruct((B, TOPK), jnp.float32)
    return pl.pallas_call(
        _boundary_body,
        out_shape=[out, out, out],
        scratch_shapes=[pltpu.VMEM((B, T), jnp.float32)],
    )(c, w, o)


def kernel(x, saliency, center_w, center_b, window_w, window_b,
           offset_w, offset_b):
    w8 = jnp.zeros((8, D), jnp.float32)
    w8 = w8.at[0].set(center_w[:, 0]).at[1].set(window_w[:, 0])
    w8 = w8.at[2].set(offset_w[:, 0])
    b8 = jnp.zeros((8, 1), jnp.float32)
    b8 = b8.at[0, 0].set(center_b[0]).at[1, 0].set(window_b[0])
    b8 = b8.at[2, 0].set(offset_b[0])
    c, w, o = _project(x, saliency, w8, b8)
    b0, b1, sc = _boundary(c.reshape(B, T), w.reshape(B, T), o.reshape(B, T))
    return jnp.stack([b0, b1, sc], axis=-1)


# speedup vs baseline: 1.6580x; 1.0964x over previous
import functools
import jax
import jax.numpy as jnp
from jax import lax
from jax.experimental import pallas as pl
from jax.experimental.pallas import tpu as pltpu

B, T, D = 16, 20000, 128
KERNEL = 5
TOPK = 100
UNIT = 2.0
TB = 800
NT = T // TB


RB = 8192
NR = -(-(B * T) // RB)


def _proj_body(w_ref, b_ref, x_ref, sal_ref, c_ref, win_ref, off_ref):
    xr = x_ref[:]
    w = w_ref[:]
    y = lax.dot_general(w, xr, (((1,), (1,)), ((), ())),
                        precision=lax.Precision.DEFAULT,
                        preferred_element_type=jnp.float32)
    y = y + b_ref[:]
    mask = jnp.where(sal_ref[:] >= 0.0, 1.0, 0.0)
    c_ref[:] = jax.nn.sigmoid(y[0]) * mask
    win_ref[:] = y[1]
    off_ref[:] = y[2]


@jax.jit
def _project(x, sal, w8, b8):
    out = jax.ShapeDtypeStruct((B * T,), jnp.float32)
    return pl.pallas_call(
        _proj_body,
        grid=(NR,),
        in_specs=[
            pl.BlockSpec((8, D), lambda r: (0, 0)),
            pl.BlockSpec((8, 1), lambda r: (0, 0)),
            pl.BlockSpec((RB, D), lambda r: (r, 0)),
            pl.BlockSpec((RB,), lambda r: (r,)),
        ],
        out_specs=[
            pl.BlockSpec((RB,), lambda r: (r,)),
            pl.BlockSpec((RB,), lambda r: (r,)),
            pl.BlockSpec((RB,), lambda r: (r,)),
        ],
        out_shape=[out, out, out],
    )(w8, b8, x.reshape(B * T, D), sal.reshape(B * T))


def _shift(a, s, fill):
    if s == 0:
        return a
    if s > 0:
        return jnp.concatenate(
            [a[:, s:], jnp.full((B, s), fill, a.dtype)], axis=1)
    return jnp.concatenate(
        [jnp.full((B, -s), fill, a.dtype), a[:, :s]], axis=1)


def _boundary_body(c_ref, w_ref, o_ref, b0_ref, b1_ref, sc_ref, p_ref):
    c = c_ref[:]
    hm = c
    for s in (-2, -1, 1, 2):
        hm = jnp.maximum(hm, _shift(c, s, -jnp.inf))
    p_ref[:] = jnp.where(hm == c, c, 0.0)

    wv = w_ref[:]
    ov = o_ref[:]

    def step(r, carry):
        b0a, b1a, sca = carry
        iota = lax.broadcasted_iota(jnp.int32, (B, T), 1)
        iota_k = lax.broadcasted_iota(jnp.int32, (1, TOPK), 1)
        p = p_ref[:]
        m = jnp.max(p, axis=1, keepdims=True)
        cand = jnp.where(p == m, iota, T)
        idx = jnp.min(cand, axis=1, keepdims=True)
        onehot = iota == idx
        off = jnp.sum(jnp.where(onehot, ov, 0.0), axis=1, keepdims=True)
        win = jnp.sum(jnp.where(onehot, wv, 0.0), axis=1, keepdims=True)
        p_ref[:] = jnp.where(onehot, -1.0, p)
        center = jnp.clip(idx.astype(jnp.float32) + off, 0.0, T - 1)
        win = jnp.clip(win, 0.0, None)
        b0 = jnp.clip(center - win * 0.5, 0.0, T - 1) * UNIT
        b1 = jnp.clip(center + win * 0.5, 0.0, T - 1) * UNIT + UNIT
        sel = (iota_k == r).astype(jnp.float32)
        return (b0a + b0 * sel, b1a + b1 * sel, sca + m * sel)

    z = jnp.zeros((B, TOPK), jnp.float32)
    b0a, b1a, sca = lax.fori_loop(0, TOPK, step, (z, z, z))
    b0_ref[:] = b0a
    b1_ref[:] = b1a
    sc_ref[:] = sca


@jax.jit
def _boundary(c, w, o):
    out = jax.ShapeDtypeStruct((B, TOPK), jnp.float32)
    return pl.pallas_call(
        _boundary_body,
        out_shape=[out, out, out],
        scratch_shapes=[pltpu.VMEM((B, T), jnp.float32)],
    )(c, w, o)


def kernel(x, saliency, center_w, center_b, window_w, window_b,
           offset_w, offset_b):
    w8 = jnp.zeros((8, D), jnp.float32)
    w8 = w8.at[0].set(center_w[:, 0]).at[1].set(window_w[:, 0])
    w8 = w8.at[2].set(offset_w[:, 0])
    b8 = jnp.zeros((8, 1), jnp.float32)
    b8 = b8.at[0, 0].set(center_b[0]).at[1, 0].set(window_b[0])
    b8 = b8.at[2, 0].set(offset_b[0])
    c, w, o = _project(x, saliency, w8, b8)
    b0, b1, sc = _boundary(c.reshape(B, T), w.reshape(B, T), o.reshape(B, T))
    return jnp.stack([b0, b1, sc], axis=-1)


# speedup vs baseline: 1.6666x; 1.0051x over previous
import jax
import jax.numpy as jnp
from jax import lax
from jax.experimental import pallas as pl
from jax.experimental.pallas import tpu as pltpu

B, T, D = 16, 20000, 128
TOPK = 100
UNIT = 2.0
RB = 8192
NR = -(-(B * T) // RB)


def _proj_body(w_ref, b_ref, x_ref, sal_ref, c_ref, win_ref, off_ref):
    xr = x_ref[:]
    w = w_ref[:]
    y = lax.dot_general(w, xr, (((1,), (1,)), ((), ())),
                        precision=lax.Precision.DEFAULT,
                        preferred_element_type=jnp.float32)
    y = y + b_ref[:]
    mask = jnp.where(sal_ref[:] >= 0.0, 1.0, 0.0)
    c_ref[:] = jax.nn.sigmoid(y[0]) * mask
    win_ref[:] = y[1]
    off_ref[:] = y[2]


@jax.jit
def _project(x, sal, w8, b8):
    out = jax.ShapeDtypeStruct((B * T,), jnp.float32)
    return pl.pallas_call(
        _proj_body,
        grid=(NR,),
        in_specs=[
            pl.BlockSpec((8, D), lambda r: (0, 0)),
            pl.BlockSpec((8, 1), lambda r: (0, 0)),
            pl.BlockSpec((RB, D), lambda r: (r, 0)),
            pl.BlockSpec((RB,), lambda r: (r,)),
        ],
        out_specs=[
            pl.BlockSpec((RB,), lambda r: (r,)),
            pl.BlockSpec((RB,), lambda r: (r,)),
            pl.BlockSpec((RB,), lambda r: (r,)),
        ],
        out_shape=[out, out, out],
    )(w8, b8, x.reshape(B * T, D), sal.reshape(B * T))


def _shift(a, s, fill):
    if s == 0:
        return a
    if s > 0:
        return jnp.concatenate(
            [a[:, s:], jnp.full((B, s), fill, a.dtype)], axis=1)
    return jnp.concatenate(
        [jnp.full((B, -s), fill, a.dtype), a[:, :s]], axis=1)


def _boundary_body(c_ref, w_ref, o_ref, b0_ref, b1_ref, sc_ref, p_ref):
    c = c_ref[:]
    hm = c
    for s in (-2, -1, 1, 2):
        hm = jnp.maximum(hm, _shift(c, s, -jnp.inf))
    p_ref[:] = jnp.where(hm == c, c, 0.0)

    wv = w_ref[:]
    ov = o_ref[:]

    def step(r, carry):
        b0a, b1a, sca = carry
        iota = lax.broadcasted_iota(jnp.int32, (B, T), 1)
        iota_k = lax.broadcasted_iota(jnp.int32, (1, TOPK), 1)
        p = p_ref[:]
        m = jnp.max(p, axis=1, keepdims=True)
        cand = jnp.where(p == m, iota, T)
        idx = jnp.min(cand, axis=1, keepdims=True)
        onehot = iota == idx
        off = jnp.sum(jnp.where(onehot, ov, 0.0), axis=1, keepdims=True)
        win = jnp.sum(jnp.where(onehot, wv, 0.0), axis=1, keepdims=True)
        p_ref[:] = jnp.where(onehot, -1.0, p)
        center = jnp.clip(idx.astype(jnp.float32) + off, 0.0, T - 1)
        win = jnp.clip(win, 0.0, None)
        b0 = jnp.clip(center - win * 0.5, 0.0, T - 1) * UNIT
        b1 = jnp.clip(center + win * 0.5, 0.0, T - 1) * UNIT + UNIT
        sel = (iota_k == r).astype(jnp.float32)
        return (b0a + b0 * sel, b1a + b1 * sel, sca + m * sel)

    z = jnp.zeros((B, TOPK), jnp.float32)
    b0a, b1a, sca = lax.fori_loop(0, TOPK, step, (z, z, z))
    b0_ref[:] = b0a
    b1_ref[:] = b1a
    sc_ref[:] = sca


@jax.jit
def _boundary(c, w, o):
    out = jax.ShapeDtypeStruct((B, TOPK), jnp.float32)
    return pl.pallas_call(
        _boundary_body,
        out_shape=[out, out, out],
        scratch_shapes=[pltpu.VMEM((B, T), jnp.float32)],
    )(c, w, o)


def kernel(x, saliency, center_w, center_b, window_w, window_b,
           offset_w, offset_b):
    w8 = jnp.zeros((8, D), jnp.float32)
    w8 = w8.at[0].set(center_w[:, 0]).at[1].set(window_w[:, 0])
    w8 = w8.at[2].set(offset_w[:, 0])
    b8 = jnp.zeros((8, 1), jnp.float32)
    b8 = b8.at[0, 0].set(center_b[0]).at[1, 0].set(window_b[0])
    b8 = b8.at[2, 0].set(offset_b[0])
    c, w, o = _project(x, saliency, w8, b8)
    b0, b1, sc = _boundary(c.reshape(B, T), w.reshape(B, T), o.reshape(B, T))
    return jnp.stack([b0, b1, sc], axis=-1)
